# baseline (speedup 1.0000x reference)
import functools
import jax
import jax.numpy as jnp
from jax import lax
from jax.experimental import pallas as pl
from jax.experimental.pallas import tpu as pltpu
from jax.experimental.pallas import tpu_sc as plsc


def _make_sc_gather(V, D, B, S):
    info = plsc.get_sparse_core_info()
    NC, NS = info.num_cores, info.num_subcores
    NW = NC * NS
    bt = B // NW
    st = S // NW
    mesh = plsc.VectorSubcoreMesh(core_axis_name="c", subcore_axis_name="s")

    @functools.partial(
        pl.kernel,
        mesh=mesh,
        out_type=(
            jax.ShapeDtypeStruct((B, D), jnp.float32),
            jax.ShapeDtypeStruct((B,), jnp.float32),
            jax.ShapeDtypeStruct((S, D), jnp.float32),
            jax.ShapeDtypeStruct((S,), jnp.float32),
        ),
        scratch_types=[
            pltpu.VMEM((bt,), jnp.int32),
            pltpu.VMEM((st,), jnp.int32),
            pltpu.VMEM((bt, D), jnp.float32),
            pltpu.VMEM((bt,), jnp.float32),
            pltpu.VMEM((st, D), jnp.float32),
            pltpu.VMEM((st,), jnp.float32),
            pltpu.SemaphoreType.DMA,
        ],
    )
    def sc_gather(lab_hbm, sid_hbm, w_hbm, b_hbm,
                  tw_out, tb_out, sw_out, sb_out,
                  lab_v, sid_v, tw_v, tb_v, sw_v, sb_v, sem):
        wid = lax.axis_index("s") * NC + lax.axis_index("c")
        lb = wid * bt
        sb = wid * st
        pltpu.sync_copy(lab_hbm.at[pl.ds(lb, bt)], lab_v)
        pltpu.sync_copy(sid_hbm.at[pl.ds(sb, st)], sid_v)
        c1 = pltpu.async_copy(w_hbm.at[lab_v], tw_v, sem)
        c2 = pltpu.async_copy(b_hbm.at[lab_v], tb_v, sem)
        c3 = pltpu.async_copy(w_hbm.at[sid_v], sw_v, sem)
        c4 = pltpu.async_copy(b_hbm.at[sid_v], sb_v, sem)
        c1.wait()
        c2.wait()
        c3.wait()
        c4.wait()
        pltpu.sync_copy(tw_v, tw_out.at[pl.ds(lb, bt)])
        pltpu.sync_copy(tb_v, tb_out.at[pl.ds(lb, bt)])
        pltpu.sync_copy(sw_v, sw_out.at[pl.ds(sb, st)])
        pltpu.sync_copy(sb_v, sb_out.at[pl.ds(sb, st)])

    return sc_gather


def _tc_body(V, S, x_ref, tw_ref, tb_ref, lab_ref, swp_ref, sbp_ref, sidp_ref,
             out_ref):
    j = pl.program_id(0)
    logvp1 = jnp.log(jnp.float32(V) + 1.0)
    ns = jnp.float32(S)

    x = x_ref[...]
    wj = swp_ref[pl.ds(j * 128, 128), :]
    v = lax.dot_general(x, wj, (((1,), (1,)), ((), ())),
                        preferred_element_type=jnp.float32)
    v = v + sbp_ref[:, pl.ds(j * 128, 128)]
    sidj = sidp_ref[:, pl.ds(j * 128, 128)]
    hits = lab_ref[...] == sidj
    v = jnp.where(hits, jnp.float32(-1e37), v)
    sidf = sidj.astype(jnp.float32)
    s_freq = (jnp.log(sidf + 2.0) - jnp.log(sidf + 1.0)) / logvp1 * ns
    v = v - jnp.log(s_freq)

    @pl.when(j == 0)
    def _():
        tl = jnp.sum(x * tw_ref[...], axis=1, keepdims=True) + tb_ref[...]
        labf = lab_ref[...].astype(jnp.float32)
        t_freq = (jnp.log(labf + 2.0) - jnp.log(labf + 1.0)) / logvp1 * ns
        tl = tl - jnp.log(t_freq)
        lane0 = lax.broadcasted_iota(jnp.int32, v.shape, 1) == 0
        out_ref[...] = jnp.where(lane0, tl, v)

    @pl.when(j != 0)
    def _():
        out_ref[...] = v

    return


def _make_tc_epilogue(V, D, B, S):
    body = functools.partial(_tc_body, V, S)
    SP = S + 128
    nj = (S + 1 + 127) // 128
    return pl.pallas_call(
        body,
        grid=(nj,),
        in_specs=[
            pl.BlockSpec((B, D), lambda j: (0, 0)),
            pl.BlockSpec((B, D), lambda j: (0, 0)),
            pl.BlockSpec((B, 1), lambda j: (0, 0)),
            pl.BlockSpec((B, 1), lambda j: (0, 0)),
            pl.BlockSpec((SP, D), lambda j: (0, 0)),
            pl.BlockSpec((1, SP), lambda j: (0, 0)),
            pl.BlockSpec((1, SP), lambda j: (0, 0)),
        ],
        out_specs=pl.BlockSpec((B, 128), lambda j: (0, j)),
        out_shape=jax.ShapeDtypeStruct((B, S + 1), jnp.float32),
    )


def kernel(inputs, labels, sample_ids, W, b):
    B, D = inputs.shape
    V = W.shape[0]
    S = sample_ids.shape[0]
    labels32 = labels.astype(jnp.int32)
    sids32 = sample_ids.astype(jnp.int32)

    tw, tb, sw, sb = _make_sc_gather(V, D, B, S)(labels32, sids32, W, b)

    swp = jnp.pad(sw, ((1, 127), (0, 0)))
    sbp = jnp.pad(sb, (1, 127))
    sidp = jnp.pad(sids32, (1, 127))

    logits = _make_tc_epilogue(V, D, B, S)(
        inputs, tw, tb[:, None], labels32[:, None], swp, sbp[None, :],
        sidp[None, :])

    new_targets = jnp.zeros((B,), dtype=jnp.int64)
    return logits, new_targets


# speedup vs baseline: 1.7009x; 1.7009x over previous
import functools
import jax
import jax.numpy as jnp
from jax import lax
from jax.experimental import pallas as pl
from jax.experimental.pallas import tpu as pltpu
from jax.experimental.pallas import tpu_sc as plsc


def _make_sc_gather(V, D, B, S):
    info = plsc.get_sparse_core_info()
    NC, NS = info.num_cores, info.num_subcores
    NW = NC * NS
    bt = B // NW
    st = S // NW
    mesh = plsc.VectorSubcoreMesh(core_axis_name="c", subcore_axis_name="s")

    @functools.partial(
        pl.kernel,
        mesh=mesh,
        out_type=(
            jax.ShapeDtypeStruct((B, D), jnp.float32),
            jax.ShapeDtypeStruct((B,), jnp.float32),
            jax.ShapeDtypeStruct((S, D), jnp.float32),
            jax.ShapeDtypeStruct((S,), jnp.float32),
        ),
        scratch_types=[
            pltpu.VMEM((bt,), jnp.int32),
            pltpu.VMEM((st,), jnp.int32),
            pltpu.VMEM((bt, D), jnp.float32),
            pltpu.VMEM((bt,), jnp.float32),
            pltpu.VMEM((st, D), jnp.float32),
            pltpu.VMEM((st,), jnp.float32),
            pltpu.SemaphoreType.DMA,
        ],
    )
    def sc_gather(lab_hbm, sid_hbm, w_hbm, b_hbm,
                  tw_out, tb_out, sw_out, sb_out,
                  lab_v, sid_v, tw_v, tb_v, sw_v, sb_v, sem):
        wid = lax.axis_index("s") * NC + lax.axis_index("c")
        lb = wid * bt
        sb = wid * st
        pltpu.sync_copy(lab_hbm.at[pl.ds(lb, bt)], lab_v)
        pltpu.sync_copy(sid_hbm.at[pl.ds(sb, st)], sid_v)
        c1 = pltpu.async_copy(w_hbm.at[lab_v], tw_v, sem)
        c2 = pltpu.async_copy(b_hbm.at[lab_v], tb_v, sem)
        c3 = pltpu.async_copy(w_hbm.at[sid_v], sw_v, sem)
        c4 = pltpu.async_copy(b_hbm.at[sid_v], sb_v, sem)
        c1.wait()
        c2.wait()
        c3.wait()
        c4.wait()
        pltpu.sync_copy(tw_v, tw_out.at[pl.ds(lb, bt)])
        pltpu.sync_copy(tb_v, tb_out.at[pl.ds(lb, bt)])
        pltpu.sync_copy(sw_v, sw_out.at[pl.ds(sb, st)])
        pltpu.sync_copy(sb_v, sb_out.at[pl.ds(sb, st)])

    return sc_gather


def _tc_body(V, S, x_ref, tw_ref, tb_ref, lab_ref, swp_ref, sbp_ref, sidp_ref,
             out_ref):
    j = pl.program_id(0)
    logvp1 = jnp.log(jnp.float32(V) + 1.0)
    ns = jnp.float32(S)

    x = x_ref[...]
    wj = swp_ref[pl.ds(j * 128, 128), :]
    v = lax.dot_general(wj, x, (((1,), (1,)), ((), ())),
                        preferred_element_type=jnp.float32)
    v = v + sbp_ref[pl.ds(j * 128, 128), :]
    sidj = sidp_ref[pl.ds(j * 128, 128), :]
    hits = sidj == lab_ref[...]
    v = jnp.where(hits, jnp.float32(-1e37), v)
    sidf = sidj.astype(jnp.float32)
    s_freq = (jnp.log(sidf + 2.0) - jnp.log(sidf + 1.0)) / logvp1 * ns
    v = v - jnp.log(s_freq)

    @pl.when(j == 0)
    def _():
        xtw = x * tw_ref[...]
        ones = jnp.ones((1, x.shape[1]), jnp.float32)
        tl = lax.dot_general(ones, xtw, (((1,), (1,)), ((), ())),
                             preferred_element_type=jnp.float32)
        tl = tl + tb_ref[...]
        labf = lab_ref[...].astype(jnp.float32)
        t_freq = (jnp.log(labf + 2.0) - jnp.log(labf + 1.0)) / logvp1 * ns
        tl = tl - jnp.log(t_freq)
        row0 = lax.broadcasted_iota(jnp.int32, v.shape, 0) == 0
        out_ref[...] = jnp.where(row0, tl, v)

    @pl.when(j != 0)
    def _():
        out_ref[...] = v

    return


def _make_tc_epilogue(V, D, B, S):
    body = functools.partial(_tc_body, V, S)
    SP = S + 128
    nj = (S + 1 + 127) // 128
    return pl.pallas_call(
        body,
        grid=(nj,),
        in_specs=[
            pl.BlockSpec((B, D), lambda j: (0, 0)),
            pl.BlockSpec((B, D), lambda j: (0, 0)),
            pl.BlockSpec((1, B), lambda j: (0, 0)),
            pl.BlockSpec((1, B), lambda j: (0, 0)),
            pl.BlockSpec((SP, D), lambda j: (0, 0)),
            pl.BlockSpec((SP, 1), lambda j: (0, 0)),
            pl.BlockSpec((SP, 1), lambda j: (0, 0)),
        ],
        out_specs=pl.BlockSpec((128, B), lambda j: (j, 0)),
        out_shape=jax.ShapeDtypeStruct((S + 1, B), jnp.float32),
    )


def kernel(inputs, labels, sample_ids, W, b):
    B, D = inputs.shape
    V = W.shape[0]
    S = sample_ids.shape[0]
    labels32 = labels.astype(jnp.int32)
    sids32 = sample_ids.astype(jnp.int32)

    tw, tb, sw, sb = _make_sc_gather(V, D, B, S)(labels32, sids32, W, b)

    swp = jnp.pad(sw, ((1, 127), (0, 0)))
    sbp = jnp.pad(sb, (1, 127))
    sidp = jnp.pad(sids32, (1, 127))

    logits_t = _make_tc_epilogue(V, D, B, S)(
        inputs, tw, tb[None, :], labels32[None, :], swp, sbp[:, None],
        sidp[:, None])

    new_targets = jnp.zeros((B,), dtype=jnp.int64)
    return logits_t.T, new_targets
